# baseline (speedup 1.0000x reference)
import functools
import numpy as np
import jax
import jax.numpy as jnp
from jax import lax
from jax.experimental import pallas as pl
from jax.experimental.pallas import tpu as pltpu
from jax.experimental.pallas import tpu_sc as plsc

SR = 16000
WIN = 400
HOP = 160
NFFT = 512
NMELS = 80
NQUANT = 256
L_ENC = 320
L_DEC = 2047

B = 16
T = 16384
NFRAMES = 1 + (T - WIN) // HOP
TDEC = T - 2 * L_ENC
NBINS = NFFT // 2 + 1

QH = 128
TCOL = 128
NCH = TDEC // TCOL
NBUF = 6


def _mel_fb_np():
    def h2m(f):
        return 2595.0 * np.log10(1.0 + f / 700.0)

    def m2h(m):
        return 700.0 * (10.0 ** (m / 2595.0) - 1.0)

    pts = np.linspace(h2m(0.0), h2m(SR / 2.0), NMELS + 2)
    hz = m2h(pts)
    bins = np.floor((NFFT + 1) * hz / SR).astype(int)
    fb = np.zeros((NMELS, NBINS), dtype=np.float32)
    for i in range(1, NMELS + 1):
        l, c, r = bins[i - 1], bins[i], bins[i + 1]
        for j in range(l, c):
            fb[i - 1, j] = (j - l) / max(c - l, 1)
        for j in range(c, min(r, NBINS)):
            fb[i - 1, j] = (r - j) / max(r - c, 1)
    return fb


def _dft_mats_np():
    w = np.hanning(WIN).astype(np.float64)
    n = np.arange(WIN, dtype=np.float64)
    k = np.arange(NBINS, dtype=np.float64)
    ang = 2.0 * np.pi * np.outer(n, k) / NFFT
    cr = np.cos(ang) * w[:, None]
    ci = np.sin(ang) * w[:, None]
    crp = np.zeros((3 * HOP, NBINS))
    cip = np.zeros((3 * HOP, NBINS))
    crp[:WIN] = cr
    cip[:WIN] = ci
    return (crp.reshape(3, HOP, NBINS).astype(np.float32),
            cip.reshape(3, HOP, NBINS).astype(np.float32))


_FB_NP = _mel_fb_np()
_WR_NP, _WI_NP = _dft_mats_np()


def _tc_body(wav3_ref, wr_ref, wi_ref, fb_ref, wavd_ref, mels_ref, code_ref):
    mu = NQUANT - 1
    x = wavd_ref[0]
    xc = jnp.clip(x, -1.0, 1.0)
    amp = jnp.sign(xc) * jnp.log1p(mu * jnp.abs(xc)) / np.log1p(mu)
    code_ref[0] = jnp.floor((amp + 1.0) * 0.5 * mu + 0.5).astype(jnp.int32)

    a = wav3_ref[0]
    a0 = a[0:NFRAMES]
    a1 = a[1:NFRAMES + 1]
    a2 = a[2:NFRAMES + 2]
    f32 = jnp.float32
    re = (jnp.dot(a0, wr_ref[0], preferred_element_type=f32)
          + jnp.dot(a1, wr_ref[1], preferred_element_type=f32)
          + jnp.dot(a2, wr_ref[2], preferred_element_type=f32))
    im = (jnp.dot(a0, wi_ref[0], preferred_element_type=f32)
          + jnp.dot(a1, wi_ref[1], preferred_element_type=f32)
          + jnp.dot(a2, wi_ref[2], preferred_element_type=f32))
    spec = re * re + im * im
    melt = lax.dot_general(fb_ref[...], spec,
                           (((1,), (1,)), ((), ())),
                           preferred_element_type=f32)
    mels_ref[0] = jnp.log(melt + 1e-6)


def _sc_onehot_body(codes_hbm, zeros_hbm, oh_hbm,
                    codes_v, bufs, sems):
    wid = lax.axis_index("s") * 2 + lax.axis_index("c")
    b = wid // 2
    q0 = (wid % 2) * QH
    ones_v = jnp.full((16,), 1.0, jnp.float32)
    zeros_v = jnp.zeros((16,), jnp.float32)

    pltpu.sync_copy(codes_hbm.at[b], codes_v)
    for p in range(NBUF):
        pltpu.sync_copy(zeros_hbm, bufs[p])

    def scatter(buf, c, vals):
        for j in range(TCOL // 16):
            cj = codes_v[pl.ds(c * TCOL + 16 * j, 16)]
            cjl = cj - q0
            m = (cjl >= 0) & (cjl < QH)
            cjc = jnp.clip(cjl, 0, QH - 1)
            tj = lax.iota(jnp.int32, 16) + (16 * j)
            plsc.store_scatter(buf, [cjc, tj], vals, mask=m)

    def dst(c):
        return oh_hbm.at[b, pl.ds(q0, QH), pl.ds(c * TCOL, TCOL)]

    def chunk(p, c):
        @pl.when(c >= NBUF)
        def _():
            pltpu.make_async_copy(bufs[p], dst(c - NBUF), sems[p]).wait()
            scatter(bufs[p], c - NBUF, zeros_v)

        scatter(bufs[p], c, ones_v)
        pltpu.async_copy(bufs[p], dst(c), sems[p])

    def body(i, carry):
        for p in range(NBUF):
            chunk(p, NBUF * i + p)
        return carry

    lax.fori_loop(0, NCH // NBUF, body, 0)
    for c in range(NCH - NCH % NBUF, NCH):
        chunk(c % NBUF, c)
    for p in range(NBUF):
        last = NCH - 1 - (NCH - 1 - p) % NBUF
        pltpu.make_async_copy(bufs[p], dst(last), sems[p]).wait()


def kernel(inds_np, wav_np, quant_onehot):
    wav3 = wav_np[:, :102 * HOP].reshape(B, 102, HOP)
    wav_dec = lax.slice(wav_np, (0, L_ENC), (B, T - L_ENC)).reshape(B, 1, TDEC)

    mels, codes = pl.pallas_call(
        _tc_body,
        grid=(B,),
        in_specs=[
            pl.BlockSpec((1, 102, HOP), lambda b: (b, 0, 0)),
            pl.BlockSpec((3, HOP, NBINS), lambda b: (0, 0, 0)),
            pl.BlockSpec((3, HOP, NBINS), lambda b: (0, 0, 0)),
            pl.BlockSpec((NMELS, NBINS), lambda b: (0, 0)),
            pl.BlockSpec((1, 1, TDEC), lambda b: (b, 0, 0)),
        ],
        out_specs=[
            pl.BlockSpec((1, NMELS, NFRAMES), lambda b: (b, 0, 0)),
            pl.BlockSpec((1, 1, TDEC), lambda b: (b, 0, 0)),
        ],
        out_shape=[
            jax.ShapeDtypeStruct((B, NMELS, NFRAMES), jnp.float32),
            jax.ShapeDtypeStruct((B, 1, TDEC), jnp.int32),
        ],
    )(wav3, jnp.asarray(_WR_NP), jnp.asarray(_WI_NP), jnp.asarray(_FB_NP),
      wav_dec)
    codes2 = codes.reshape(B, TDEC)

    mesh = plsc.VectorSubcoreMesh(core_axis_name="c", subcore_axis_name="s")
    sc_onehot = functools.partial(
        pl.kernel,
        mesh=mesh,
        out_type=jax.ShapeDtypeStruct((B, NQUANT, TDEC), jnp.float32),
        scratch_types=[
            pltpu.VMEM((TDEC,), jnp.int32),
            [pltpu.VMEM((QH, TCOL), jnp.float32) for _ in range(NBUF)],
            [pltpu.SemaphoreType.DMA for _ in range(NBUF)],
        ],
        compiler_params=pltpu.CompilerParams(needs_layout_passes=False),
    )(_sc_onehot_body)
    onehot = sc_onehot(codes2, jnp.zeros((QH, TCOL), jnp.float32))

    wav_compand_out = lax.slice(codes2, (0, L_DEC), (B, TDEC))
    return (inds_np, mels, onehot, wav_compand_out)


# speedup vs baseline: 1.0943x; 1.0943x over previous
import functools
import numpy as np
import jax
import jax.numpy as jnp
from jax import lax
from jax.experimental import pallas as pl
from jax.experimental.pallas import tpu as pltpu
from jax.experimental.pallas import tpu_sc as plsc

SR = 16000
WIN = 400
HOP = 160
NFFT = 512
NMELS = 80
NQUANT = 256
L_ENC = 320
L_DEC = 2047

B = 16
T = 16384
NFRAMES = 1 + (T - WIN) // HOP
TDEC = T - 2 * L_ENC
NBINS = NFFT // 2 + 1

QH = 128
TCOL = 128
NCH = TDEC // TCOL


def _mel_fb_np():
    def h2m(f):
        return 2595.0 * np.log10(1.0 + f / 700.0)

    def m2h(m):
        return 700.0 * (10.0 ** (m / 2595.0) - 1.0)

    pts = np.linspace(h2m(0.0), h2m(SR / 2.0), NMELS + 2)
    hz = m2h(pts)
    bins = np.floor((NFFT + 1) * hz / SR).astype(int)
    fb = np.zeros((NMELS, NBINS), dtype=np.float32)
    for i in range(1, NMELS + 1):
        l, c, r = bins[i - 1], bins[i], bins[i + 1]
        for j in range(l, c):
            fb[i - 1, j] = (j - l) / max(c - l, 1)
        for j in range(c, min(r, NBINS)):
            fb[i - 1, j] = (r - j) / max(r - c, 1)
    return fb


def _dft_mats_np():
    w = np.hanning(WIN).astype(np.float64)
    n = np.arange(WIN, dtype=np.float64)
    k = np.arange(NBINS, dtype=np.float64)
    ang = 2.0 * np.pi * np.outer(n, k) / NFFT
    cr = np.cos(ang) * w[:, None]
    ci = np.sin(ang) * w[:, None]
    crp = np.zeros((3 * HOP, NBINS))
    cip = np.zeros((3 * HOP, NBINS))
    crp[:WIN] = cr
    cip[:WIN] = ci
    return (crp.reshape(3, HOP, NBINS).astype(np.float32),
            cip.reshape(3, HOP, NBINS).astype(np.float32))


_FB_NP = _mel_fb_np()
_WR_NP, _WI_NP = _dft_mats_np()


def _tc_body(wav3_ref, wr_ref, wi_ref, fb_ref, wavd_ref, mels_ref, code_ref):
    mu = NQUANT - 1
    x = wavd_ref[0]
    xc = jnp.clip(x, -1.0, 1.0)
    amp = jnp.sign(xc) * jnp.log1p(mu * jnp.abs(xc)) / np.log1p(mu)
    code_ref[0] = jnp.floor((amp + 1.0) * 0.5 * mu + 0.5).astype(jnp.int32)

    a = wav3_ref[0]
    a0 = a[0:NFRAMES]
    a1 = a[1:NFRAMES + 1]
    a2 = a[2:NFRAMES + 2]
    f32 = jnp.float32
    re = (jnp.dot(a0, wr_ref[0], preferred_element_type=f32)
          + jnp.dot(a1, wr_ref[1], preferred_element_type=f32)
          + jnp.dot(a2, wr_ref[2], preferred_element_type=f32))
    im = (jnp.dot(a0, wi_ref[0], preferred_element_type=f32)
          + jnp.dot(a1, wi_ref[1], preferred_element_type=f32)
          + jnp.dot(a2, wi_ref[2], preferred_element_type=f32))
    spec = re * re + im * im
    melt = lax.dot_general(fb_ref[...], spec,
                           (((1,), (1,)), ((), ())),
                           preferred_element_type=f32)
    mels_ref[0] = jnp.log(melt + 1e-6)


def _sc_onehot_body(codes_hbm, zeros_hbm, oh_hbm,
                    codes_v, buf0, buf1, sem0, sem1):
    wid = lax.axis_index("s") * 2 + lax.axis_index("c")
    b = wid // 2
    q0 = (wid % 2) * QH
    ones_v = jnp.full((16,), 1.0, jnp.float32)
    zeros_v = jnp.zeros((16,), jnp.float32)

    pltpu.sync_copy(codes_hbm.at[b], codes_v)
    pltpu.sync_copy(zeros_hbm, buf0)
    pltpu.sync_copy(zeros_hbm, buf1)

    def scatter(buf, c, vals):
        for j in range(TCOL // 16):
            cj = codes_v[pl.ds(c * TCOL + 16 * j, 16)]
            cjl = cj - q0
            m = (cjl >= 0) & (cjl < QH)
            cjc = jnp.clip(cjl, 0, QH - 1)
            tj = lax.iota(jnp.int32, 16) + (16 * j)
            plsc.store_scatter(buf, [cjc, tj], vals, mask=m)

    def dst(c):
        return oh_hbm.at[b, pl.ds(q0, QH), pl.ds(c * TCOL, TCOL)]

    def chunk(buf, sem, c):
        @pl.when(c >= 2)
        def _():
            pltpu.make_async_copy(buf, dst(c - 2), sem).wait()
            scatter(buf, c - 2, zeros_v)

        scatter(buf, c, ones_v)
        pltpu.async_copy(buf, dst(c), sem)

    def body2(i, carry):
        chunk(buf0, sem0, 2 * i)
        chunk(buf1, sem1, 2 * i + 1)
        return carry

    lax.fori_loop(0, NCH // 2, body2, 0)
    chunk(buf0, sem0, NCH - 1)
    pltpu.make_async_copy(buf1, dst(NCH - 2), sem1).wait()
    pltpu.make_async_copy(buf0, dst(NCH - 1), sem0).wait()


def kernel(inds_np, wav_np, quant_onehot):
    wav3 = wav_np[:, :102 * HOP].reshape(B, 102, HOP)
    wav_dec = lax.slice(wav_np, (0, L_ENC), (B, T - L_ENC)).reshape(B, 1, TDEC)
    mels, codes = pl.pallas_call(
        _tc_body,
        grid=(B,),
        in_specs=[
            pl.BlockSpec((1, 102, HOP), lambda b: (b, 0, 0)),
            pl.BlockSpec((3, HOP, NBINS), lambda b: (0, 0, 0)),
            pl.BlockSpec((3, HOP, NBINS), lambda b: (0, 0, 0)),
            pl.BlockSpec((NMELS, NBINS), lambda b: (0, 0)),
            pl.BlockSpec((1, 1, TDEC), lambda b: (b, 0, 0)),
        ],
        out_specs=[
            pl.BlockSpec((1, NMELS, NFRAMES), lambda b: (b, 0, 0)),
            pl.BlockSpec((1, 1, TDEC), lambda b: (b, 0, 0)),
        ],
        out_shape=[
            jax.ShapeDtypeStruct((B, NMELS, NFRAMES), jnp.float32),
            jax.ShapeDtypeStruct((B, 1, TDEC), jnp.int32),
        ],
    )(wav3, jnp.asarray(_WR_NP), jnp.asarray(_WI_NP), jnp.asarray(_FB_NP),
      wav_dec)
    codes2 = codes.reshape(B, TDEC)

    mesh = plsc.VectorSubcoreMesh(core_axis_name="c", subcore_axis_name="s")
    sc_onehot = functools.partial(
        pl.kernel,
        mesh=mesh,
        out_type=jax.ShapeDtypeStruct((B, NQUANT, TDEC), jnp.float32),
        scratch_types=[
            pltpu.VMEM((TDEC,), jnp.int32),
            pltpu.VMEM((QH, TCOL), jnp.float32),
            pltpu.VMEM((QH, TCOL), jnp.float32),
            pltpu.SemaphoreType.DMA,
            pltpu.SemaphoreType.DMA,
        ],
        compiler_params=pltpu.CompilerParams(needs_layout_passes=False),
    )(_sc_onehot_body)
    onehot = sc_onehot(codes2, jnp.zeros((QH, TCOL), jnp.float32))

    wav_compand_out = lax.slice(codes2, (0, L_DEC), (B, TDEC))
    return (inds_np, mels, onehot, wav_compand_out)


# speedup vs baseline: 1.1473x; 1.0484x over previous
import functools
import numpy as np
import jax
import jax.numpy as jnp
from jax import lax
from jax.experimental import pallas as pl
from jax.experimental.pallas import tpu as pltpu
from jax.experimental.pallas import tpu_sc as plsc

SR = 16000
WIN = 400
HOP = 160
NFFT = 512
NMELS = 80
NQUANT = 256
L_ENC = 320
L_DEC = 2047

B = 16
T = 16384
NFRAMES = 1 + (T - WIN) // HOP
TDEC = T - 2 * L_ENC
NBINS = NFFT // 2 + 1

QH = 128
TCOL = 128
NCH = TDEC // TCOL
NBUF = 2


def _mel_fb_np():
    def h2m(f):
        return 2595.0 * np.log10(1.0 + f / 700.0)

    def m2h(m):
        return 700.0 * (10.0 ** (m / 2595.0) - 1.0)

    pts = np.linspace(h2m(0.0), h2m(SR / 2.0), NMELS + 2)
    hz = m2h(pts)
    bins = np.floor((NFFT + 1) * hz / SR).astype(int)
    fb = np.zeros((NMELS, NBINS), dtype=np.float32)
    for i in range(1, NMELS + 1):
        l, c, r = bins[i - 1], bins[i], bins[i + 1]
        for j in range(l, c):
            fb[i - 1, j] = (j - l) / max(c - l, 1)
        for j in range(c, min(r, NBINS)):
            fb[i - 1, j] = (r - j) / max(r - c, 1)
    return fb


def _dft_mats_np():
    w = np.hanning(WIN).astype(np.float64)
    n = np.arange(WIN, dtype=np.float64)
    k = np.arange(NBINS, dtype=np.float64)
    ang = 2.0 * np.pi * np.outer(n, k) / NFFT
    cr = np.cos(ang) * w[:, None]
    ci = np.sin(ang) * w[:, None]
    crp = np.zeros((3 * HOP, NBINS))
    cip = np.zeros((3 * HOP, NBINS))
    crp[:WIN] = cr
    cip[:WIN] = ci
    return (crp.reshape(3, HOP, NBINS).astype(np.float32),
            cip.reshape(3, HOP, NBINS).astype(np.float32))


_FB_NP = _mel_fb_np()
_WR_NP, _WI_NP = _dft_mats_np()


def _codes_body(wavd_ref, code_ref):
    mu = NQUANT - 1
    x = wavd_ref[0]
    xc = jnp.clip(x, -1.0, 1.0)
    amp = jnp.sign(xc) * jnp.log1p(mu * jnp.abs(xc)) / np.log1p(mu)
    code_ref[0] = jnp.floor((amp + 1.0) * 0.5 * mu + 0.5).astype(jnp.int32)


def _mels_body(wav3_ref, wr_ref, wi_ref, fb_ref, mels_ref):
    a = wav3_ref[0]
    a0 = a[0:NFRAMES]
    a1 = a[1:NFRAMES + 1]
    a2 = a[2:NFRAMES + 2]
    f32 = jnp.float32
    re = (jnp.dot(a0, wr_ref[0], preferred_element_type=f32)
          + jnp.dot(a1, wr_ref[1], preferred_element_type=f32)
          + jnp.dot(a2, wr_ref[2], preferred_element_type=f32))
    im = (jnp.dot(a0, wi_ref[0], preferred_element_type=f32)
          + jnp.dot(a1, wi_ref[1], preferred_element_type=f32)
          + jnp.dot(a2, wi_ref[2], preferred_element_type=f32))
    spec = re * re + im * im
    melt = lax.dot_general(fb_ref[...], spec,
                           (((1,), (1,)), ((), ())),
                           preferred_element_type=f32)
    mels_ref[0] = jnp.log(melt + 1e-6)


def _sc_onehot_body(codes_hbm, zeros_hbm, oh_hbm,
                    codes_v, bufs, sems):
    wid = lax.axis_index("s") * 2 + lax.axis_index("c")
    b = wid // 2
    q0 = (wid % 2) * QH
    ones_v = jnp.full((16,), 1.0, jnp.float32)
    zeros_v = jnp.zeros((16,), jnp.float32)

    pltpu.sync_copy(codes_hbm.at[b], codes_v)
    for p in range(NBUF):
        pltpu.sync_copy(zeros_hbm, bufs[p])

    def scatter(buf, c, vals):
        for j in range(TCOL // 16):
            cj = codes_v[pl.ds(c * TCOL + 16 * j, 16)]
            cjl = cj - q0
            m = (cjl >= 0) & (cjl < QH)
            cjc = jnp.clip(cjl, 0, QH - 1)
            tj = lax.iota(jnp.int32, 16) + (16 * j)
            plsc.store_scatter(buf, [cjc, tj], vals, mask=m)

    def dst(c):
        return oh_hbm.at[b, pl.ds(q0, QH), pl.ds(c * TCOL, TCOL)]

    def chunk(p, c):
        @pl.when(c >= NBUF)
        def _():
            pltpu.make_async_copy(bufs[p], dst(c - NBUF), sems[p]).wait()
            scatter(bufs[p], c - NBUF, zeros_v)

        scatter(bufs[p], c, ones_v)
        pltpu.async_copy(bufs[p], dst(c), sems[p])

    def body(i, carry):
        for p in range(NBUF):
            chunk(p, NBUF * i + p)
        return carry

    lax.fori_loop(0, NCH // NBUF, body, 0)
    for c in range(NCH - NCH % NBUF, NCH):
        chunk(c % NBUF, c)
    for p in range(NBUF):
        last = NCH - 1 - (NCH - 1 - p) % NBUF
        pltpu.make_async_copy(bufs[p], dst(last), sems[p]).wait()


def kernel(inds_np, wav_np, quant_onehot):
    wav3 = wav_np[:, :102 * HOP].reshape(B, 102, HOP)
    wav_dec = lax.slice(wav_np, (0, L_ENC), (B, T - L_ENC)).reshape(B, 1, TDEC)

    codes = pl.pallas_call(
        _codes_body,
        grid=(B,),
        in_specs=[pl.BlockSpec((1, 1, TDEC), lambda b: (b, 0, 0))],
        out_specs=pl.BlockSpec((1, 1, TDEC), lambda b: (b, 0, 0)),
        out_shape=jax.ShapeDtypeStruct((B, 1, TDEC), jnp.int32),
    )(wav_dec)
    codes2 = codes.reshape(B, TDEC)

    mesh = plsc.VectorSubcoreMesh(core_axis_name="c", subcore_axis_name="s")
    sc_onehot = functools.partial(
        pl.kernel,
        mesh=mesh,
        out_type=jax.ShapeDtypeStruct((B, NQUANT, TDEC), jnp.float32),
        scratch_types=[
            pltpu.VMEM((TDEC,), jnp.int32),
            [pltpu.VMEM((QH, TCOL), jnp.float32) for _ in range(NBUF)],
            [pltpu.SemaphoreType.DMA for _ in range(NBUF)],
        ],
        compiler_params=pltpu.CompilerParams(needs_layout_passes=False),
    )(_sc_onehot_body)
    onehot = sc_onehot(codes2, jnp.zeros((QH, TCOL), jnp.float32))

    mels = pl.pallas_call(
        _mels_body,
        grid=(B,),
        in_specs=[
            pl.BlockSpec((1, 102, HOP), lambda b: (b, 0, 0)),
            pl.BlockSpec((3, HOP, NBINS), lambda b: (0, 0, 0)),
            pl.BlockSpec((3, HOP, NBINS), lambda b: (0, 0, 0)),
            pl.BlockSpec((NMELS, NBINS), lambda b: (0, 0)),
        ],
        out_specs=pl.BlockSpec((1, NMELS, NFRAMES), lambda b: (b, 0, 0)),
        out_shape=jax.ShapeDtypeStruct((B, NMELS, NFRAMES), jnp.float32),
    )(wav3, jnp.asarray(_WR_NP), jnp.asarray(_WI_NP), jnp.asarray(_FB_NP))

    wav_compand_out = lax.slice(codes2, (0, L_DEC), (B, TDEC))
    return (inds_np, mels, onehot, wav_compand_out)
